# baseline (speedup 1.0000x reference)
import jax
import jax.numpy as jnp
from jax.experimental import pallas as pl
from jax.experimental.pallas import tpu as pltpu

B = 4096; S = 50; PF = 16; SD = 128; BF = 64
D = 256; ADK = 64; HID = 128; C = 20; NDIR = 3; NACT = 4
NSLOT = 128 // PF


def _fused_kernel(price_ref, sent_ref, beh_ref, ids_ref,
                  W8_ref, bp8_ref, Wp2_ref, bp2_ref, Ws_ref, bs_ref, Wb_ref, bb_ref,
                  attn_W_ref, attn_b_ref, attn_v_ref,
                  WdT_ref, bd_ref, WuT_ref, bu_ref,
                  W1T_ref, b1_ref, Wcomb_ref, bcomb_ref, E64_ref, E128_ref,
                  fused_ref, out8_ref, G_ref, *, n_s):
    i = pl.program_id(0)
    bf = jnp.bfloat16
    nrow = price_ref.shape[0]
    bt = nrow * NSLOT // n_s

    @pl.when(i == 0)
    def _():
        r_iota = jax.lax.broadcasted_iota(jnp.int32, (bt, nrow), 1)
        b_iota = jax.lax.broadcasted_iota(jnp.int32, (bt, nrow), 0)
        for l in range(NSLOT):
            sel = (NSLOT * r_iota + l) // n_s == b_iota
            G_ref[l] = sel.astype(bf)

    xp = price_ref[...].astype(bf)
    z8 = jnp.dot(xp, W8_ref[...], preferred_element_type=jnp.float32)
    zb = jnp.maximum(z8 + bp8_ref[...], 0.0).astype(bf)
    pm = jnp.dot(G_ref[0], zb[:, :D], preferred_element_type=jnp.float32)
    for l in range(1, NSLOT):
        pm = pm + jnp.dot(G_ref[l], zb[:, l * D:(l + 1) * D],
                          preferred_element_type=jnp.float32)
    pm = pm * (1.0 / n_s)

    pe = jnp.maximum(jnp.dot(pm.astype(bf), Wp2_ref[...],
                             preferred_element_type=jnp.float32) + bp2_ref[...], 0.0)
    se = jnp.maximum(jnp.dot(sent_ref[...], Ws_ref[...],
                             preferred_element_type=jnp.float32) + bs_ref[...], 0.0)
    be = jnp.maximum(jnp.dot(beh_ref[...], Wb_ref[...],
                             preferred_element_type=jnp.float32) + bb_ref[...], 0.0)

    aW = attn_W_ref[...]; ab = attn_b_ref[...]; av = attn_v_ref[...]

    def score(u):
        t = jnp.tanh(jnp.dot(u.astype(bf), aW, preferred_element_type=jnp.float32) + ab)
        return jnp.dot(t.astype(bf), av, preferred_element_type=jnp.float32)

    s0 = score(pe); s1 = score(se); s2 = score(be)
    m = jnp.maximum(jnp.maximum(s0, s1), s2)
    e0 = jnp.exp(s0 - m); e1 = jnp.exp(s1 - m); e2 = jnp.exp(s2 - m)
    inv = 1.0 / (e0 + e1 + e2)
    fused = (e0 * inv) * pe + (e1 * inv) * se + (e2 * inv) * be
    fused_ref[...] = fused

    ids = ids_ref[...]
    onehot = (ids == jax.lax.broadcasted_iota(jnp.int32, (bt, C), 1)
              ).astype(bf)
    mask_a = jnp.dot(onehot, E64_ref[...], preferred_element_type=jnp.float32)
    mask_h = jnp.dot(onehot, E128_ref[...], preferred_element_type=jnp.float32)

    z1 = jnp.dot(fused.astype(bf), WdT_ref[...],
                 preferred_element_type=jnp.float32) + bd_ref[...]
    h_mask = jnp.maximum(z1, 0.0) * mask_a
    bu_tok = jnp.dot(onehot, bu_ref[...], preferred_element_type=jnp.float32)
    adapted = fused + jnp.dot(h_mask.astype(bf), WuT_ref[...],
                              preferred_element_type=jnp.float32) + bu_tok

    z2 = jnp.dot(adapted.astype(bf), W1T_ref[...],
                 preferred_element_type=jnp.float32) + b1_ref[...]
    t_mask = jnp.maximum(z2, 0.0) * mask_h
    out8 = jnp.dot(t_mask.astype(bf), Wcomb_ref[...], preferred_element_type=jnp.float32) \
        + jnp.dot(onehot, bcomb_ref[...], preferred_element_type=jnp.float32)
    out8_ref[...] = out8


def kernel(price, sentiment, behavior, client_ids, Wp, bp, Wp2, bp2, Ws, bs, Wb, bb,
           attn_W, attn_b, attn_v, A_Wd, A_bd, A_Wu, A_bu,
           H_W1, H_b1, H_Wdir, H_bdir, H_Wrisk, H_brisk, H_Wact, H_bact):
    b, n_s, pf = price.shape
    d = Wp.shape[1]
    c, _, adk = A_Wd.shape
    hid = H_W1.shape[2]
    bf = jnp.bfloat16
    bt = min(256, b)
    nb = b // bt
    nrow = bt * n_s * pf // 128

    price_p = price.reshape(b * n_s * pf // 128, 128)
    ids2d = client_ids.reshape(b, 1)

    W8 = jnp.kron(jnp.eye(NSLOT, dtype=Wp.dtype), Wp).astype(bf)
    bp8 = jnp.tile(bp, NSLOT).reshape(1, NSLOT * d)

    WdT = jnp.transpose(A_Wd, (1, 0, 2)).reshape(d, c * adk).astype(bf)
    bd_flat = A_bd.reshape(1, c * adk)
    WuT = A_Wu.reshape(c * adk, d).astype(bf)
    W1T = jnp.transpose(H_W1, (1, 0, 2)).reshape(d, c * hid).astype(bf)
    b1_flat = H_b1.reshape(1, c * hid)
    Wcomb = jnp.concatenate([H_Wdir, H_Wrisk[..., None], H_Wact], axis=2)
    Wcomb = Wcomb.reshape(c * hid, NDIR + 1 + NACT).astype(bf)
    bcomb = jnp.concatenate([H_bdir, H_brisk[:, None], H_bact], axis=1)
    E64 = jnp.kron(jnp.eye(c, dtype=jnp.float32), jnp.ones((1, adk))).astype(bf)
    E128 = jnp.kron(jnp.eye(c, dtype=jnp.float32), jnp.ones((1, hid))).astype(bf)

    full = lambda shape: pl.BlockSpec(shape, lambda i: (0,) * len(shape))
    grid = (nb,)

    fused, out8 = pl.pallas_call(
        lambda *refs: _fused_kernel(*refs, n_s=n_s),
        grid=grid,
        in_specs=[
            pl.BlockSpec((nrow, 128), lambda i: (i, 0)),
            pl.BlockSpec((bt, SD), lambda i: (i, 0)),
            pl.BlockSpec((bt, BF), lambda i: (i, 0)),
            pl.BlockSpec((bt, 1), lambda i: (i, 0)),
            full((128, NSLOT * d)), full((1, NSLOT * d)),
            full((d, d)), full((1, d)),
            full((SD, d)), full((1, d)),
            full((BF, d)), full((1, d)),
            full((d, hid)), full((1, hid)), full((hid, 1)),
            full((d, c * adk)), full((1, c * adk)),
            full((c * adk, d)), full((c, d)),
            full((d, c * hid)), full((1, c * hid)),
            full((c * hid, 8)), full((c, 8)),
            full((c, c * adk)), full((c, c * hid)),
        ],
        out_specs=[
            pl.BlockSpec((bt, d), lambda i: (i, 0)),
            pl.BlockSpec((bt, 8), lambda i: (i, 0)),
        ],
        out_shape=[
            jax.ShapeDtypeStruct((b, d), jnp.float32),
            jax.ShapeDtypeStruct((b, 8), jnp.float32),
        ],
        scratch_shapes=[pltpu.VMEM((NSLOT, bt, nrow), bf)],
    )(price_p, sentiment.astype(bf), behavior.astype(bf), ids2d,
      W8, bp8, Wp2.astype(bf), bp2.reshape(1, d),
      Ws.astype(bf), bs.reshape(1, d), Wb.astype(bf), bb.reshape(1, d),
      attn_W.astype(bf), attn_b.reshape(1, hid), attn_v.reshape(hid, 1).astype(bf),
      WdT, bd_flat, WuT, A_bu, W1T, b1_flat, Wcomb, bcomb, E64, E128)

    direction = out8[:, :NDIR]
    risk = out8[:, NDIR]
    action = out8[:, NDIR + 1:NDIR + 1 + NACT]
    return direction, risk, action, fused


# speedup vs baseline: 1.5262x; 1.5262x over previous
import jax
import jax.numpy as jnp
from jax.experimental import pallas as pl
from jax.experimental.pallas import tpu as pltpu

B = 4096; S = 50; PF = 16; SD = 128; BF = 64
D = 256; ADK = 64; HID = 128; C = 20; NDIR = 3; NACT = 4


def _fused_kernel(price_ref, sent_ref, beh_ref, ids_ref,
                  Wp_ref, bp_ref, Wp2_ref, bp2_ref, Ws_ref, bs_ref, Wb_ref, bb_ref,
                  attn_W_ref, attn_b_ref, attn_v_ref,
                  WdT_ref, bd_ref, WuT_ref, bu_ref,
                  W1T_ref, b1_ref, Wcomb_ref, bcomb_ref, E64_ref, E128_ref,
                  fused_ref, out8_ref, *, n_s):
    bf = jnp.bfloat16
    x3 = price_ref[...]
    n_s2, bt, pf = x3.shape
    x = x3.reshape(n_s2 * bt, pf)
    z = jnp.dot(x, Wp_ref[...], preferred_element_type=jnp.float32) + bp_ref[...]
    ph = jnp.maximum(z, 0.0)
    pm = jnp.sum(ph.reshape(n_s2, bt, ph.shape[1]), axis=0) * (1.0 / n_s)

    pe = jnp.maximum(jnp.dot(pm.astype(bf), Wp2_ref[...],
                             preferred_element_type=jnp.float32) + bp2_ref[...], 0.0)
    se = jnp.maximum(jnp.dot(sent_ref[...], Ws_ref[...],
                             preferred_element_type=jnp.float32) + bs_ref[...], 0.0)
    be = jnp.maximum(jnp.dot(beh_ref[...], Wb_ref[...],
                             preferred_element_type=jnp.float32) + bb_ref[...], 0.0)

    aW = attn_W_ref[...]; ab = attn_b_ref[...]; av = attn_v_ref[...]

    def score(u):
        t = jnp.tanh(jnp.dot(u.astype(bf), aW, preferred_element_type=jnp.float32) + ab)
        return jnp.dot(t.astype(bf), av, preferred_element_type=jnp.float32)

    s0 = score(pe); s1 = score(se); s2 = score(be)
    m = jnp.maximum(jnp.maximum(s0, s1), s2)
    e0 = jnp.exp(s0 - m); e1 = jnp.exp(s1 - m); e2 = jnp.exp(s2 - m)
    inv = 1.0 / (e0 + e1 + e2)
    fused = (e0 * inv) * pe + (e1 * inv) * se + (e2 * inv) * be
    fused_ref[...] = fused

    ids = ids_ref[...]
    onehot = (ids == jax.lax.broadcasted_iota(jnp.int32, (bt, C), 1)
              ).astype(bf)
    mask_a = jnp.dot(onehot, E64_ref[...], preferred_element_type=jnp.float32)
    mask_h = jnp.dot(onehot, E128_ref[...], preferred_element_type=jnp.float32)

    z1 = jnp.dot(fused.astype(bf), WdT_ref[...],
                 preferred_element_type=jnp.float32) + bd_ref[...]
    h_mask = jnp.maximum(z1, 0.0) * mask_a
    bu_tok = jnp.dot(onehot, bu_ref[...], preferred_element_type=jnp.float32)
    adapted = fused + jnp.dot(h_mask.astype(bf), WuT_ref[...],
                              preferred_element_type=jnp.float32) + bu_tok

    z2 = jnp.dot(adapted.astype(bf), W1T_ref[...],
                 preferred_element_type=jnp.float32) + b1_ref[...]
    t_mask = jnp.maximum(z2, 0.0) * mask_h
    out8 = jnp.dot(t_mask.astype(bf), Wcomb_ref[...], preferred_element_type=jnp.float32) \
        + jnp.dot(onehot, bcomb_ref[...], preferred_element_type=jnp.float32)
    out8_ref[...] = out8


def kernel(price, sentiment, behavior, client_ids, Wp, bp, Wp2, bp2, Ws, bs, Wb, bb,
           attn_W, attn_b, attn_v, A_Wd, A_bd, A_Wu, A_bu,
           H_W1, H_b1, H_Wdir, H_bdir, H_Wrisk, H_brisk, H_Wact, H_bact):
    b, n_s, pf = price.shape
    d = Wp.shape[1]
    c, _, adk = A_Wd.shape
    hid = H_W1.shape[2]
    bf = jnp.bfloat16
    bt = min(256, b)
    nb = b // bt

    price_t = jnp.transpose(price, (1, 0, 2)).astype(bf)
    ids2d = client_ids.reshape(b, 1)

    WdT = jnp.transpose(A_Wd, (1, 0, 2)).reshape(d, c * adk).astype(bf)
    bd_flat = A_bd.reshape(1, c * adk)
    WuT = A_Wu.reshape(c * adk, d).astype(bf)
    W1T = jnp.transpose(H_W1, (1, 0, 2)).reshape(d, c * hid).astype(bf)
    b1_flat = H_b1.reshape(1, c * hid)
    Wcomb = jnp.concatenate([H_Wdir, H_Wrisk[..., None], H_Wact], axis=2)
    Wcomb = Wcomb.reshape(c * hid, NDIR + 1 + NACT).astype(bf)
    bcomb = jnp.concatenate([H_bdir, H_brisk[:, None], H_bact], axis=1)
    E64 = jnp.kron(jnp.eye(c, dtype=jnp.float32), jnp.ones((1, adk))).astype(bf)
    E128 = jnp.kron(jnp.eye(c, dtype=jnp.float32), jnp.ones((1, hid))).astype(bf)

    full = lambda shape: pl.BlockSpec(shape, lambda i: (0,) * len(shape))
    grid = (nb,)

    fused, out8 = pl.pallas_call(
        lambda *refs: _fused_kernel(*refs, n_s=n_s),
        grid=grid,
        in_specs=[
            pl.BlockSpec((n_s, bt, pf), lambda i: (0, i, 0)),
            pl.BlockSpec((bt, SD), lambda i: (i, 0)),
            pl.BlockSpec((bt, BF), lambda i: (i, 0)),
            pl.BlockSpec((bt, 1), lambda i: (i, 0)),
            full((pf, d)), full((1, d)),
            full((d, d)), full((1, d)),
            full((SD, d)), full((1, d)),
            full((BF, d)), full((1, d)),
            full((d, hid)), full((1, hid)), full((hid, 1)),
            full((d, c * adk)), full((1, c * adk)),
            full((c * adk, d)), full((c, d)),
            full((d, c * hid)), full((1, c * hid)),
            full((c * hid, 8)), full((c, 8)),
            full((c, c * adk)), full((c, c * hid)),
        ],
        out_specs=[
            pl.BlockSpec((bt, d), lambda i: (i, 0)),
            pl.BlockSpec((bt, 8), lambda i: (i, 0)),
        ],
        out_shape=[
            jax.ShapeDtypeStruct((b, d), jnp.float32),
            jax.ShapeDtypeStruct((b, 8), jnp.float32),
        ],
    )(price_t, sentiment.astype(bf), behavior.astype(bf), ids2d,
      Wp.astype(bf), bp.reshape(1, d), Wp2.astype(bf), bp2.reshape(1, d),
      Ws.astype(bf), bs.reshape(1, d), Wb.astype(bf), bb.reshape(1, d),
      attn_W.astype(bf), attn_b.reshape(1, hid), attn_v.reshape(hid, 1).astype(bf),
      WdT, bd_flat, WuT, A_bu, W1T, b1_flat, Wcomb, bcomb, E64, E128)

    direction = out8[:, :NDIR]
    risk = out8[:, NDIR]
    action = out8[:, NDIR + 1:NDIR + 1 + NACT]
    return direction, risk, action, fused


# speedup vs baseline: 1.6789x; 1.1001x over previous
import jax
import jax.numpy as jnp
from jax.experimental import pallas as pl
from jax.experimental.pallas import tpu as pltpu

B = 4096; S = 50; PF = 16; SD = 128; BF = 64
D = 256; ADK = 64; HID = 128; C = 20; NDIR = 3; NACT = 4


def _fused_kernel(price_ref, sent_ref, beh_ref, ids_ref,
                  Wp_ref, bp_ref, Wp2_ref, bp2_ref, Ws_ref, bs_ref, Wb_ref, bb_ref,
                  attn_W_ref, attn_b_ref, attn_v_ref,
                  WdT_ref, bd_ref, WuT_ref, bu_ref,
                  W1T_ref, b1_ref, Wcomb_ref, bcomb_ref,
                  fused_ref, out8_ref, *, n_s):
    bf = jnp.bfloat16
    x3 = price_ref[...]
    n_s2, bt, pf = x3.shape
    x = x3.reshape(n_s2 * bt, pf)
    z = jnp.dot(x, Wp_ref[...], preferred_element_type=jnp.float32) + bp_ref[...]
    ph = jnp.maximum(z, 0.0)
    pm = jnp.sum(ph.reshape(n_s2, bt, ph.shape[1]), axis=0) * (1.0 / n_s)

    pe = jnp.maximum(jnp.dot(pm.astype(bf), Wp2_ref[...],
                             preferred_element_type=jnp.float32) + bp2_ref[...], 0.0)
    se = jnp.maximum(jnp.dot(sent_ref[...], Ws_ref[...],
                             preferred_element_type=jnp.float32) + bs_ref[...], 0.0)
    be = jnp.maximum(jnp.dot(beh_ref[...], Wb_ref[...],
                             preferred_element_type=jnp.float32) + bb_ref[...], 0.0)

    aW = attn_W_ref[...]; ab = attn_b_ref[...]; av = attn_v_ref[...]

    def score(u):
        t = jnp.tanh(jnp.dot(u.astype(bf), aW, preferred_element_type=jnp.float32) + ab)
        return jnp.dot(t.astype(bf), av, preferred_element_type=jnp.float32)

    s0 = score(pe); s1 = score(se); s2 = score(be)
    m = jnp.maximum(jnp.maximum(s0, s1), s2)
    e0 = jnp.exp(s0 - m); e1 = jnp.exp(s1 - m); e2 = jnp.exp(s2 - m)
    inv = 1.0 / (e0 + e1 + e2)
    fused = (e0 * inv) * pe + (e1 * inv) * se + (e2 * inv) * be
    fused_ref[...] = fused

    ids = ids_ref[...]
    onehot = (ids == jax.lax.broadcasted_iota(jnp.int32, (bt, C), 1)
              ).astype(jnp.float32)
    lane_a = jax.lax.broadcasted_iota(jnp.int32, (bt, C * ADK), 1)
    mask_a = (lane_a >> 6) == ids
    lane_h = jax.lax.broadcasted_iota(jnp.int32, (bt, C * HID), 1)
    mask_h = (lane_h >> 7) == ids

    z1 = jnp.dot(fused.astype(bf), WdT_ref[...],
                 preferred_element_type=jnp.float32) + bd_ref[...]
    h_mask = jnp.where(mask_a, jnp.maximum(z1, 0.0), 0.0)
    bu_tok = jnp.dot(onehot, bu_ref[...], preferred_element_type=jnp.float32)
    adapted = fused + jnp.dot(h_mask.astype(bf), WuT_ref[...],
                              preferred_element_type=jnp.float32) + bu_tok

    z2 = jnp.dot(adapted.astype(bf), W1T_ref[...],
                 preferred_element_type=jnp.float32) + b1_ref[...]
    t_mask = jnp.where(mask_h, jnp.maximum(z2, 0.0), 0.0)
    out8 = jnp.dot(t_mask.astype(bf), Wcomb_ref[...], preferred_element_type=jnp.float32) \
        + jnp.dot(onehot, bcomb_ref[...], preferred_element_type=jnp.float32)
    out8_ref[...] = out8


def kernel(price, sentiment, behavior, client_ids, Wp, bp, Wp2, bp2, Ws, bs, Wb, bb,
           attn_W, attn_b, attn_v, A_Wd, A_bd, A_Wu, A_bu,
           H_W1, H_b1, H_Wdir, H_bdir, H_Wrisk, H_brisk, H_Wact, H_bact):
    b, n_s, pf = price.shape
    d = Wp.shape[1]
    c, _, adk = A_Wd.shape
    hid = H_W1.shape[2]
    bf = jnp.bfloat16
    bt = min(256, b)
    nb = b // bt

    price_t = jnp.transpose(price, (1, 0, 2)).astype(bf)
    ids2d = client_ids.reshape(b, 1)

    WdT = jnp.transpose(A_Wd, (1, 0, 2)).reshape(d, c * adk).astype(bf)
    bd_flat = A_bd.reshape(1, c * adk)
    WuT = A_Wu.reshape(c * adk, d).astype(bf)
    W1T = jnp.transpose(H_W1, (1, 0, 2)).reshape(d, c * hid).astype(bf)
    b1_flat = H_b1.reshape(1, c * hid)
    Wcomb = jnp.concatenate([H_Wdir, H_Wrisk[..., None], H_Wact], axis=2)
    Wcomb = Wcomb.reshape(c * hid, NDIR + 1 + NACT).astype(bf)
    bcomb = jnp.concatenate([H_bdir, H_brisk[:, None], H_bact], axis=1)

    full = lambda shape: pl.BlockSpec(shape, lambda i: (0,) * len(shape))
    grid = (nb,)

    fused, out8 = pl.pallas_call(
        lambda *refs: _fused_kernel(*refs, n_s=n_s),
        grid=grid,
        in_specs=[
            pl.BlockSpec((n_s, bt, pf), lambda i: (0, i, 0)),
            pl.BlockSpec((bt, SD), lambda i: (i, 0)),
            pl.BlockSpec((bt, BF), lambda i: (i, 0)),
            pl.BlockSpec((bt, 1), lambda i: (i, 0)),
            full((pf, d)), full((1, d)),
            full((d, d)), full((1, d)),
            full((SD, d)), full((1, d)),
            full((BF, d)), full((1, d)),
            full((d, hid)), full((1, hid)), full((hid, 1)),
            full((d, c * adk)), full((1, c * adk)),
            full((c * adk, d)), full((c, d)),
            full((d, c * hid)), full((1, c * hid)),
            full((c * hid, 8)), full((c, 8)),
        ],
        out_specs=[
            pl.BlockSpec((bt, d), lambda i: (i, 0)),
            pl.BlockSpec((bt, 8), lambda i: (i, 0)),
        ],
        out_shape=[
            jax.ShapeDtypeStruct((b, d), jnp.float32),
            jax.ShapeDtypeStruct((b, 8), jnp.float32),
        ],
    )(price_t, sentiment.astype(bf), behavior.astype(bf), ids2d,
      Wp.astype(bf), bp.reshape(1, d), Wp2.astype(bf), bp2.reshape(1, d),
      Ws.astype(bf), bs.reshape(1, d), Wb.astype(bf), bb.reshape(1, d),
      attn_W.astype(bf), attn_b.reshape(1, hid), attn_v.reshape(hid, 1).astype(bf),
      WdT, bd_flat, WuT, A_bu, W1T, b1_flat, Wcomb, bcomb)

    direction = out8[:, :NDIR]
    risk = out8[:, NDIR]
    action = out8[:, NDIR + 1:NDIR + 1 + NACT]
    return direction, risk, action, fused


# speedup vs baseline: 1.7600x; 1.0483x over previous
import jax
import jax.numpy as jnp
from jax.experimental import pallas as pl
from jax.experimental.pallas import tpu as pltpu

B = 4096; S = 50; PF = 16; SD = 128; BF = 64
D = 256; ADK = 64; HID = 128; C = 20; NDIR = 3; NACT = 4


def _fused_kernel(price_ref, sent_ref, beh_ref, ids_ref,
                  Wp_ref, bp_ref, Wp2_ref, bp2_ref, Ws_ref, bs_ref, Wb_ref, bb_ref,
                  attn_W_ref, attn_b_ref, attn_v_ref,
                  WdT_ref, bd_ref, WuT_ref, bu_ref,
                  W1T_ref, b1_ref, Wcomb_ref, bcomb_ref,
                  fused_ref, out8_ref, *, n_s):
    bf = jnp.bfloat16
    x3 = price_ref[...]
    n_s2, bt, pf = x3.shape
    x = x3.reshape(n_s2 * bt, pf)
    z = jnp.dot(x, Wp_ref[...], preferred_element_type=jnp.float32) + bp_ref[...]
    ph = jnp.maximum(z, 0.0)
    pm = jnp.sum(ph.reshape(n_s2, bt, ph.shape[1]), axis=0) * (1.0 / n_s)

    pe = jnp.maximum(jnp.dot(pm.astype(bf), Wp2_ref[...],
                             preferred_element_type=jnp.float32) + bp2_ref[...], 0.0)
    se = jnp.maximum(jnp.dot(sent_ref[...], Ws_ref[...],
                             preferred_element_type=jnp.float32) + bs_ref[...], 0.0)
    be = jnp.maximum(jnp.dot(beh_ref[...], Wb_ref[...],
                             preferred_element_type=jnp.float32) + bb_ref[...], 0.0)

    aW = attn_W_ref[...]; ab = attn_b_ref[...]; av = attn_v_ref[...]

    def score(u):
        t = jnp.tanh(jnp.dot(u.astype(bf), aW, preferred_element_type=jnp.float32) + ab)
        return jnp.dot(t.astype(bf), av, preferred_element_type=jnp.float32)

    s0 = score(pe); s1 = score(se); s2 = score(be)
    m = jnp.maximum(jnp.maximum(s0, s1), s2)
    e0 = jnp.exp(s0 - m); e1 = jnp.exp(s1 - m); e2 = jnp.exp(s2 - m)
    inv = 1.0 / (e0 + e1 + e2)
    fused = (e0 * inv) * pe + (e1 * inv) * se + (e2 * inv) * be
    fused_ref[...] = fused

    ids = ids_ref[...]
    onehot = (ids == jax.lax.broadcasted_iota(jnp.int32, (bt, C), 1)
              ).astype(jnp.float32)
    lane_a = jax.lax.broadcasted_iota(jnp.int32, (bt, C * ADK), 1)
    mask_a = (lane_a >> 6) == ids
    lane_h = jax.lax.broadcasted_iota(jnp.int32, (bt, C * HID), 1)
    mask_h = (lane_h >> 7) == ids

    z1 = jnp.dot(fused.astype(bf), WdT_ref[...],
                 preferred_element_type=jnp.float32) + bd_ref[...]
    h_mask = jnp.where(mask_a, jnp.maximum(z1, 0.0), 0.0)
    bu_tok = jnp.dot(onehot, bu_ref[...], preferred_element_type=jnp.float32)
    adapted = fused + jnp.dot(h_mask.astype(bf), WuT_ref[...],
                              preferred_element_type=jnp.float32) + bu_tok

    z2 = jnp.dot(adapted.astype(bf), W1T_ref[...],
                 preferred_element_type=jnp.float32) + b1_ref[...]
    t_mask = jnp.where(mask_h, jnp.maximum(z2, 0.0), 0.0)
    out8 = jnp.dot(t_mask.astype(bf), Wcomb_ref[...], preferred_element_type=jnp.float32) \
        + jnp.dot(onehot, bcomb_ref[...], preferred_element_type=jnp.float32)
    out8_ref[...] = out8


def kernel(price, sentiment, behavior, client_ids, Wp, bp, Wp2, bp2, Ws, bs, Wb, bb,
           attn_W, attn_b, attn_v, A_Wd, A_bd, A_Wu, A_bu,
           H_W1, H_b1, H_Wdir, H_bdir, H_Wrisk, H_brisk, H_Wact, H_bact):
    b, n_s, pf = price.shape
    d = Wp.shape[1]
    c, _, adk = A_Wd.shape
    hid = H_W1.shape[2]
    bf = jnp.bfloat16
    bt = min(512, b)
    nb = b // bt

    price_t = jnp.transpose(price, (1, 0, 2)).astype(bf)
    ids2d = client_ids.reshape(b, 1)

    WdT = jnp.transpose(A_Wd, (1, 0, 2)).reshape(d, c * adk).astype(bf)
    bd_flat = A_bd.reshape(1, c * adk)
    WuT = A_Wu.reshape(c * adk, d).astype(bf)
    W1T = jnp.transpose(H_W1, (1, 0, 2)).reshape(d, c * hid).astype(bf)
    b1_flat = H_b1.reshape(1, c * hid)
    Wcomb = jnp.concatenate([H_Wdir, H_Wrisk[..., None], H_Wact], axis=2)
    Wcomb = Wcomb.reshape(c * hid, NDIR + 1 + NACT).astype(bf)
    bcomb = jnp.concatenate([H_bdir, H_brisk[:, None], H_bact], axis=1)

    full = lambda shape: pl.BlockSpec(shape, lambda i: (0,) * len(shape))
    grid = (nb,)

    fused, out8 = pl.pallas_call(
        lambda *refs: _fused_kernel(*refs, n_s=n_s),
        grid=grid,
        in_specs=[
            pl.BlockSpec((n_s, bt, pf), lambda i: (0, i, 0)),
            pl.BlockSpec((bt, SD), lambda i: (i, 0)),
            pl.BlockSpec((bt, BF), lambda i: (i, 0)),
            pl.BlockSpec((bt, 1), lambda i: (i, 0)),
            full((pf, d)), full((1, d)),
            full((d, d)), full((1, d)),
            full((SD, d)), full((1, d)),
            full((BF, d)), full((1, d)),
            full((d, hid)), full((1, hid)), full((hid, 1)),
            full((d, c * adk)), full((1, c * adk)),
            full((c * adk, d)), full((c, d)),
            full((d, c * hid)), full((1, c * hid)),
            full((c * hid, 8)), full((c, 8)),
        ],
        out_specs=[
            pl.BlockSpec((bt, d), lambda i: (i, 0)),
            pl.BlockSpec((bt, 8), lambda i: (i, 0)),
        ],
        out_shape=[
            jax.ShapeDtypeStruct((b, d), jnp.float32),
            jax.ShapeDtypeStruct((b, 8), jnp.float32),
        ],
    )(price_t, sentiment.astype(bf), behavior.astype(bf), ids2d,
      Wp.astype(bf), bp.reshape(1, d), Wp2.astype(bf), bp2.reshape(1, d),
      Ws.astype(bf), bs.reshape(1, d), Wb.astype(bf), bb.reshape(1, d),
      attn_W.astype(bf), attn_b.reshape(1, hid), attn_v.reshape(hid, 1).astype(bf),
      WdT, bd_flat, WuT, A_bu, W1T, b1_flat, Wcomb, bcomb)

    direction = out8[:, :NDIR]
    risk = out8[:, NDIR]
    action = out8[:, NDIR + 1:NDIR + 1 + NACT]
    return direction, risk, action, fused


# speedup vs baseline: 1.7724x; 1.0070x over previous
import jax
import jax.numpy as jnp
from jax.experimental import pallas as pl
from jax.experimental.pallas import tpu as pltpu

B = 4096; S = 50; PF = 16; SD = 128; BF = 64
D = 256; ADK = 64; HID = 128; C = 20; NDIR = 3; NACT = 4


def _fused_kernel(price_ref, sent_ref, beh_ref, ids_ref,
                  Wp_ref, bp_ref, Wp2_ref, bp2_ref, Ws_ref, bs_ref, Wb_ref, bb_ref,
                  attn_W_ref, attn_b_ref, attn_v_ref,
                  WdT_ref, bd_ref, WuT_ref, bu_ref,
                  W1T_ref, b1_ref, Wcomb_ref, bcomb_ref,
                  fused_ref, out8_ref, *, n_s):
    bf = jnp.bfloat16
    x3 = price_ref[...]
    n_s2, bt, pf = x3.shape
    x = x3.reshape(n_s2 * bt, pf)
    z = jnp.dot(x, Wp_ref[...], preferred_element_type=jnp.float32) + bp_ref[...]
    ph = jnp.maximum(z, 0.0)
    pm = jnp.sum(ph.reshape(n_s2, bt, ph.shape[1]), axis=0) * (1.0 / n_s)

    pe = jnp.maximum(jnp.dot(pm.astype(bf), Wp2_ref[...],
                             preferred_element_type=jnp.float32) + bp2_ref[...], 0.0)
    se = jnp.maximum(jnp.dot(sent_ref[...], Ws_ref[...],
                             preferred_element_type=jnp.float32) + bs_ref[...], 0.0)
    be = jnp.maximum(jnp.dot(beh_ref[...], Wb_ref[...],
                             preferred_element_type=jnp.float32) + bb_ref[...], 0.0)

    aW = attn_W_ref[...]; ab = attn_b_ref[...]; av = attn_v_ref[...]

    def score(u):
        t = jnp.tanh(jnp.dot(u.astype(bf), aW, preferred_element_type=jnp.float32) + ab)
        return jnp.dot(t.astype(bf), av, preferred_element_type=jnp.float32)

    s0 = score(pe); s1 = score(se); s2 = score(be)
    m = jnp.maximum(jnp.maximum(s0, s1), s2)
    e0 = jnp.exp(s0 - m); e1 = jnp.exp(s1 - m); e2 = jnp.exp(s2 - m)
    inv = 1.0 / (e0 + e1 + e2)
    fused = (e0 * inv) * pe + (e1 * inv) * se + (e2 * inv) * be
    fused_ref[...] = fused

    ids = ids_ref[...]
    onehot = (ids == jax.lax.broadcasted_iota(jnp.int32, (bt, C), 1)
              ).astype(jnp.float32)
    lane_a = jax.lax.broadcasted_iota(jnp.int32, (bt, C * ADK), 1)
    mask_a = (lane_a >> 6) == ids
    lane_h = jax.lax.broadcasted_iota(jnp.int32, (bt, C * HID), 1)
    mask_h = (lane_h >> 7) == ids

    z1 = jnp.dot(fused.astype(bf), WdT_ref[...],
                 preferred_element_type=jnp.float32) + bd_ref[...]
    h_mask = jnp.where(mask_a, jnp.maximum(z1, 0.0), 0.0)
    bu_tok = jnp.dot(onehot, bu_ref[...], preferred_element_type=jnp.float32)
    adapted = fused + jnp.dot(h_mask.astype(bf), WuT_ref[...],
                              preferred_element_type=jnp.float32) + bu_tok

    z2 = jnp.dot(adapted.astype(bf), W1T_ref[...],
                 preferred_element_type=jnp.float32) + b1_ref[...]
    t_mask = jnp.where(mask_h, jnp.maximum(z2, 0.0), 0.0)
    out8 = jnp.dot(t_mask.astype(bf), Wcomb_ref[...], preferred_element_type=jnp.float32) \
        + jnp.dot(onehot, bcomb_ref[...], preferred_element_type=jnp.float32)
    out8_ref[...] = out8


def kernel(price, sentiment, behavior, client_ids, Wp, bp, Wp2, bp2, Ws, bs, Wb, bb,
           attn_W, attn_b, attn_v, A_Wd, A_bd, A_Wu, A_bu,
           H_W1, H_b1, H_Wdir, H_bdir, H_Wrisk, H_brisk, H_Wact, H_bact):
    b, n_s, pf = price.shape
    d = Wp.shape[1]
    c, _, adk = A_Wd.shape
    hid = H_W1.shape[2]
    bf = jnp.bfloat16
    bt = min(1024, b)
    nb = b // bt

    price_t = jnp.transpose(price, (1, 0, 2)).astype(bf)
    ids2d = client_ids.reshape(b, 1)

    WdT = jnp.transpose(A_Wd, (1, 0, 2)).reshape(d, c * adk).astype(bf)
    bd_flat = A_bd.reshape(1, c * adk)
    WuT = A_Wu.reshape(c * adk, d).astype(bf)
    W1T = jnp.transpose(H_W1, (1, 0, 2)).reshape(d, c * hid).astype(bf)
    b1_flat = H_b1.reshape(1, c * hid)
    Wcomb = jnp.concatenate([H_Wdir, H_Wrisk[..., None], H_Wact], axis=2)
    Wcomb = Wcomb.reshape(c * hid, NDIR + 1 + NACT).astype(bf)
    bcomb = jnp.concatenate([H_bdir, H_brisk[:, None], H_bact], axis=1)

    full = lambda shape: pl.BlockSpec(shape, lambda i: (0,) * len(shape))
    grid = (nb,)

    fused, out8 = pl.pallas_call(
        lambda *refs: _fused_kernel(*refs, n_s=n_s),
        grid=grid,
        in_specs=[
            pl.BlockSpec((n_s, bt, pf), lambda i: (0, i, 0)),
            pl.BlockSpec((bt, SD), lambda i: (i, 0)),
            pl.BlockSpec((bt, BF), lambda i: (i, 0)),
            pl.BlockSpec((bt, 1), lambda i: (i, 0)),
            full((pf, d)), full((1, d)),
            full((d, d)), full((1, d)),
            full((SD, d)), full((1, d)),
            full((BF, d)), full((1, d)),
            full((d, hid)), full((1, hid)), full((hid, 1)),
            full((d, c * adk)), full((1, c * adk)),
            full((c * adk, d)), full((c, d)),
            full((d, c * hid)), full((1, c * hid)),
            full((c * hid, 8)), full((c, 8)),
        ],
        out_specs=[
            pl.BlockSpec((bt, d), lambda i: (i, 0)),
            pl.BlockSpec((bt, 8), lambda i: (i, 0)),
        ],
        out_shape=[
            jax.ShapeDtypeStruct((b, d), jnp.float32),
            jax.ShapeDtypeStruct((b, 8), jnp.float32),
        ],
    )(price_t, sentiment.astype(bf), behavior.astype(bf), ids2d,
      Wp.astype(bf), bp.reshape(1, d), Wp2.astype(bf), bp2.reshape(1, d),
      Ws.astype(bf), bs.reshape(1, d), Wb.astype(bf), bb.reshape(1, d),
      attn_W.astype(bf), attn_b.reshape(1, hid), attn_v.reshape(hid, 1).astype(bf),
      WdT, bd_flat, WuT, A_bu, W1T, b1_flat, Wcomb, bcomb)

    direction = out8[:, :NDIR]
    risk = out8[:, NDIR]
    action = out8[:, NDIR + 1:NDIR + 1 + NACT]
    return direction, risk, action, fused
